# baseline (speedup 1.0000x reference)
import functools

import jax
import jax.numpy as jnp
from jax import lax
from jax.experimental import pallas as pl
from jax.experimental.pallas import tpu as pltpu
from jax.experimental.pallas import tpu_sc as plsc

NC = 2
NS = 16
NW = NC * NS
CHUNK = 128
NBUF = 8

_MESH = plsc.VectorSubcoreMesh(
    core_axis_name="c", subcore_axis_name="s", num_cores=NC, num_subcores=NS
)


@functools.lru_cache(maxsize=None)
def _make_idx_permute(batch, hist):
    n_flat = batch * hist
    b_per_w = batch // NW
    f_per_w = n_flat // NW
    n_vec = f_per_w // 16

    @functools.partial(
        pl.kernel,
        out_type=jax.ShapeDtypeStruct((NW, f_per_w), jnp.int32),
        mesh=_MESH,
        scratch_types=[
            pltpu.VMEM((hist, b_per_w), jnp.int32),
            pltpu.VMEM((f_per_w,), jnp.int32),
        ],
        compiler_params=pltpu.CompilerParams(needs_layout_passes=False),
    )
    def permute_kernel(in_hbm, out_hbm, buf_in, buf_out):
        wid = lax.axis_index("s") * NC + lax.axis_index("c")
        b0 = wid * b_per_w
        pltpu.sync_copy(in_hbm.at[:, pl.ds(b0, b_per_w)], buf_in)
        lanes = lax.iota(jnp.int32, 16)

        @pl.loop(0, n_vec, init_carry=(lanes, jnp.zeros((16,), jnp.int32)))
        def _vec(g, carry):
            h, c = carry
            vals = plsc.load_gather(buf_in, [h, c])
            buf_out[pl.ds(g * 16, 16)] = vals
            h2 = h + 16
            wrap = h2 >= hist
            h_new = jnp.where(wrap, h2 - hist, h2)
            c_new = c + wrap.astype(jnp.int32)
            return (h_new, c_new)

        pltpu.sync_copy(buf_out, out_hbm.at[wid])

    return permute_kernel


@functools.lru_cache(maxsize=None)
def _make_gather(n_rows, d):
    assert n_rows % (NW * CHUNK) == 0
    b_per_w = n_rows // NW
    n_chunks = b_per_w // CHUNK
    assert n_chunks % NBUF == 0

    @functools.partial(
        pl.kernel,
        out_type=jax.ShapeDtypeStruct((n_rows, d), jnp.float32),
        mesh=_MESH,
        scratch_types=[
            pltpu.VMEM((n_chunks, CHUNK), jnp.int32),
            pltpu.VMEM((NBUF, CHUNK, d), jnp.float32),
        ]
        + [pltpu.SemaphoreType.DMA] * (2 * NBUF),
        compiler_params=pltpu.CompilerParams(use_tc_tiling_on_sc=False),
    )
    def gather_kernel(table_hbm, idx_hbm, out_hbm, idx_v, rows_v, *sems):
        gsems = sems[:NBUF]
        wsems = sems[NBUF:]
        wid = lax.axis_index("s") * NC + lax.axis_index("c")
        base = wid * b_per_w
        pltpu.sync_copy(idx_hbm.at[wid], idx_v)

        def start_gather(b, j):
            pltpu.async_copy(table_hbm.at[idx_v.at[j]], rows_v.at[b], gsems[b])

        def wait_gather(b):
            pltpu.make_async_copy(
                table_hbm.at[pl.ds(0, CHUNK)], rows_v.at[b], gsems[b]
            ).wait()

        def start_write(b, j):
            pltpu.async_copy(
                rows_v.at[b], out_hbm.at[pl.ds(base + j * CHUNK, CHUNK)], wsems[b]
            )

        def wait_write(b):
            pltpu.make_async_copy(
                rows_v.at[b], out_hbm.at[pl.ds(0, CHUNK)], wsems[b]
            ).wait()

        for b in range(NBUF):
            start_gather(b, b)

        @pl.loop(0, n_chunks - NBUF, step=NBUF)
        def _outer(g):
            for b in range(NBUF):
                wait_gather(b)
                start_write(b, g + b)
            for b in range(NBUF):
                wait_write(b)
                start_gather(b, g + b + NBUF)

        g0 = n_chunks - NBUF
        for b in range(NBUF):
            wait_gather(b)
            start_write(b, g0 + b)
        for b in range(NBUF):
            wait_write(b)

    return gather_kernel


def kernel(embedding_matrix, inputs):
    b, h = inputs.shape
    d = embedding_matrix.shape[1]
    inputs_t = inputs.T.astype(jnp.int32)
    idx_flat = _make_idx_permute(b, h)(inputs_t)
    idx3 = idx_flat.reshape(NW, -1, CHUNK)
    out = _make_gather(b * h, d)(embedding_matrix, idx3)
    return out.reshape(b, h, d)


# speedup vs baseline: 1.5916x; 1.5916x over previous
import functools

import jax
import jax.numpy as jnp
from jax import lax
from jax.experimental import pallas as pl
from jax.experimental.pallas import tpu as pltpu
from jax.experimental.pallas import tpu_sc as plsc

NC = 2
NS = 16
NW = NC * NS
NBUF = 8
BCH = 4

_MESH = plsc.VectorSubcoreMesh(
    core_axis_name="c", subcore_axis_name="s", num_cores=NC, num_subcores=NS
)


@functools.lru_cache(maxsize=None)
def _make_lookup(batch, hist, d):
    assert batch % NW == 0
    b_per_w = batch // NW
    n_ch = b_per_w // BCH
    assert b_per_w % BCH == 0 and n_ch % NBUF == 0
    hpad = (hist + 7) // 8 * 8
    roww = BCH * hpad
    assert roww % 16 == 0
    n_grp = roww // 16

    @functools.partial(
        pl.kernel,
        out_type=jax.ShapeDtypeStruct((batch, hist, d), jnp.float32),
        mesh=_MESH,
        scratch_types=[
            pltpu.VMEM((hist, b_per_w), jnp.int32),
            pltpu.VMEM((n_ch, roww), jnp.int32),
            pltpu.VMEM((NBUF, BCH, hpad, d), jnp.float32),
        ]
        + [pltpu.SemaphoreType.DMA] * (2 * NBUF),
        compiler_params=pltpu.CompilerParams(
            use_tc_tiling_on_sc=False, needs_layout_passes=False
        ),
    )
    def lookup_kernel(table_hbm, in_hbm, out_hbm, buf_in, idx2, ring, *sems):
        gsems = sems[:NBUF]
        wsems = sems[NBUF:]
        wid = lax.axis_index("s") * NC + lax.axis_index("c")
        b0 = wid * b_per_w
        pltpu.sync_copy(in_hbm.at[:, pl.ds(b0, b_per_w)], buf_in)

        lanes = lax.iota(jnp.int32, 16)
        h_pat, boff_pat = [], []
        for g in range(n_grp):
            j = lanes + (g * 16)
            boff = jnp.zeros((16,), jnp.int32)
            for _ in range(BCH):
                wrap = j >= hpad
                j = jnp.where(wrap, j - hpad, j)
                boff = boff + wrap.astype(jnp.int32)
            h = jnp.where(j >= hist, j - hist, j)
            h_pat.append(h)
            boff_pat.append(boff)

        @pl.loop(0, n_ch, init_carry=jnp.zeros((16,), jnp.int32))
        def _row(cc, c):
            for g in range(n_grp):
                vals = plsc.load_gather(buf_in, [h_pat[g], c + boff_pat[g]])
                idx2[cc, pl.ds(g * 16, 16)] = vals
            return c + BCH

        def start_gather(slot, cc):
            for b4 in range(BCH):
                pltpu.async_copy(
                    table_hbm.at[idx2.at[cc, pl.ds(b4 * hpad, hpad)]],
                    ring.at[slot, b4],
                    gsems[slot],
                )

        def wait_gather(slot):
            for b4 in range(BCH):
                pltpu.make_async_copy(
                    table_hbm.at[pl.ds(0, hpad)],
                    ring.at[slot, b4],
                    gsems[slot],
                ).wait()

        def start_write(slot, cc):
            pltpu.async_copy(
                ring.at[slot, :, pl.ds(0, hist)],
                out_hbm.at[pl.ds(b0 + cc * BCH, BCH)],
                wsems[slot],
            )

        def wait_write(slot):
            pltpu.make_async_copy(
                ring.at[slot, :, pl.ds(0, hist)],
                out_hbm.at[pl.ds(0, BCH)],
                wsems[slot],
            ).wait()

        for s in range(NBUF):
            start_gather(s, s)

        @pl.loop(0, n_ch - NBUF, step=NBUF)
        def _outer(g):
            for s in range(NBUF):
                wait_gather(s)
                start_write(s, g + s)
            for s in range(NBUF):
                wait_write(s)
                start_gather(s, g + s + NBUF)

        g0 = n_ch - NBUF
        for s in range(NBUF):
            wait_gather(s)
            start_write(s, g0 + s)
        for s in range(NBUF):
            wait_write(s)

    return lookup_kernel


def kernel(embedding_matrix, inputs):
    b, h = inputs.shape
    d = embedding_matrix.shape[1]
    inputs_t = inputs.T.astype(jnp.int32)
    return _make_lookup(b, h, d)(embedding_matrix, inputs_t)


# speedup vs baseline: 1.6145x; 1.0144x over previous
import functools

import jax
import jax.numpy as jnp
from jax import lax
from jax.experimental import pallas as pl
from jax.experimental.pallas import tpu as pltpu
from jax.experimental.pallas import tpu_sc as plsc

NC = 2
NS = 16
NW = NC * NS
NBUF = 8
BCH = 4

_MESH = plsc.VectorSubcoreMesh(
    core_axis_name="c", subcore_axis_name="s", num_cores=NC, num_subcores=NS
)


def _patterns(hist, hpad, n_grp):
    lanes = lax.iota(jnp.int32, 16)
    h_pat, boff_pat = [], []
    for g in range(n_grp):
        j = lanes + (g * 16)
        boff = jnp.zeros((16,), jnp.int32)
        for _ in range(BCH):
            wrap = j >= hpad
            j = jnp.where(wrap, j - hpad, j)
            boff = boff + wrap.astype(jnp.int32)
        h = jnp.where(j >= hist, j - hist, j)
        h_pat.append(h)
        boff_pat.append(boff)
    return h_pat, boff_pat


@functools.lru_cache(maxsize=None)
def _make_permute(batch, hist):
    b_per_w = batch // NW
    hpad = (hist + 7) // 8 * 8
    roww = BCH * hpad
    n_ch = b_per_w // BCH
    n_grp = roww // 16
    width = n_ch * roww
    assert width % 128 == 0 and roww % 16 == 0

    @functools.partial(
        pl.kernel,
        out_type=jax.ShapeDtypeStruct((NW, width), jnp.int32),
        mesh=_MESH,
        scratch_types=[
            pltpu.VMEM((hist, b_per_w), jnp.int32),
            pltpu.VMEM((width,), jnp.int32),
        ],
        compiler_params=pltpu.CompilerParams(needs_layout_passes=False),
    )
    def permute_kernel(in_hbm, out_hbm, buf_in, buf_out):
        wid = lax.axis_index("s") * NC + lax.axis_index("c")
        b0 = wid * b_per_w
        pltpu.sync_copy(in_hbm.at[:, pl.ds(b0, b_per_w)], buf_in)
        h_pat, boff_pat = _patterns(hist, hpad, n_grp)

        @pl.loop(0, n_ch, init_carry=jnp.zeros((16,), jnp.int32))
        def _chunk(cc, c):
            for g in range(n_grp):
                vals = plsc.load_gather(buf_in, [h_pat[g], c + boff_pat[g]])
                buf_out[pl.ds(cc * roww + g * 16, 16)] = vals
            return c + BCH

        pltpu.sync_copy(buf_out, out_hbm.at[wid])

    return permute_kernel


@functools.lru_cache(maxsize=None)
def _make_gather(batch, hist, d):
    assert batch % NW == 0
    b_per_w = batch // NW
    n_ch = b_per_w // BCH
    assert b_per_w % BCH == 0 and n_ch % NBUF == 0
    hpad = (hist + 7) // 8 * 8
    roww = BCH * hpad
    width = n_ch * roww

    @functools.partial(
        pl.kernel,
        out_type=jax.ShapeDtypeStruct((batch, hist, d), jnp.float32),
        mesh=_MESH,
        scratch_types=[
            pltpu.VMEM((width,), jnp.int32),
            pltpu.VMEM((NBUF, BCH, hpad, d), jnp.float32),
        ]
        + [pltpu.SemaphoreType.DMA] * (2 * NBUF),
        compiler_params=pltpu.CompilerParams(
            use_tc_tiling_on_sc=False, needs_layout_passes=False
        ),
    )
    def gather_kernel(table_hbm, idx_hbm, out_hbm, idx_v, ring, *sems):
        gsems = sems[:NBUF]
        wsems = sems[NBUF:]
        wid = lax.axis_index("s") * NC + lax.axis_index("c")
        b0 = wid * b_per_w
        pltpu.sync_copy(idx_hbm.at[wid], idx_v)

        def start_gather(slot, cc):
            for b4 in range(BCH):
                pltpu.async_copy(
                    table_hbm.at[idx_v.at[pl.ds(cc * roww + b4 * hpad, hpad)]],
                    ring.at[slot, b4],
                    gsems[slot],
                )

        def wait_gather(slot):
            for b4 in range(BCH):
                pltpu.make_async_copy(
                    table_hbm.at[pl.ds(0, hpad)],
                    ring.at[slot, b4],
                    gsems[slot],
                ).wait()

        def start_write(slot, cc):
            pltpu.async_copy(
                ring.at[slot, :, pl.ds(0, hist)],
                out_hbm.at[pl.ds(b0 + cc * BCH, BCH)],
                wsems[slot],
            )

        def wait_write(slot):
            pltpu.make_async_copy(
                ring.at[slot, :, pl.ds(0, hist)],
                out_hbm.at[pl.ds(0, BCH)],
                wsems[slot],
            ).wait()

        for s in range(NBUF):
            start_gather(s, s)

        @pl.loop(0, n_ch - NBUF, step=NBUF)
        def _outer(g):
            for s in range(NBUF):
                wait_gather(s)
                start_write(s, g + s)
            for s in range(NBUF):
                wait_write(s)
                start_gather(s, g + s + NBUF)

        g0 = n_ch - NBUF
        for s in range(NBUF):
            wait_gather(s)
            start_write(s, g0 + s)
        for s in range(NBUF):
            wait_write(s)

    return gather_kernel


def kernel(embedding_matrix, inputs):
    b, h = inputs.shape
    d = embedding_matrix.shape[1]
    inputs_t = inputs.T.astype(jnp.int32)
    idx_rows = _make_permute(b, h)(inputs_t)
    return _make_gather(b, h, d)(embedding_matrix, idx_rows)
